# baseline (speedup 1.0000x reference)
import functools

import jax
import jax.numpy as jnp
from jax import lax
from jax.experimental import pallas as pl
from jax.experimental.pallas import tpu as pltpu
from jax.experimental.pallas import tpu_sc as plsc

_P = 16
_E = 128
_H = 256
_W = 256
_LANES = 16
_HHALF = _H // 2
_NCH = _HHALF // _LANES


def _ceil_i32(x):
  it = x.astype(jnp.int32)
  return it + (x > it.astype(jnp.float32)).astype(jnp.int32)


def _sc_hist_body(cont_hbm, hist_hbm, cont_v, comp_v, bnd_v, hist_v):
  wid = lax.axis_index("s") * 2 + lax.axis_index("c")
  p = wid // 2
  half = wid % 2
  ylo_half = half * _HHALF

  pltpu.sync_copy(cont_hbm.at[p], cont_v)

  lane = lax.iota(jnp.int32, _LANES)
  ones = jnp.ones((_LANES,), jnp.float32)
  zeros = jnp.zeros((_LANES,), jnp.float32)
  zeros_i = jnp.zeros((_LANES,), jnp.int32)
  ones_i = jnp.ones((_LANES,), jnp.int32)
  fmax = float(_W - 1)

  bnd_v[pl.ds(_E, _LANES)] = zeros_i

  def pbody(j, _):
    e0 = pl.multiple_of(j * _LANES, _LANES)
    v0 = e0 + lane
    v1 = (v0 + 1) & (_E - 1)
    gx = plsc.load_gather(cont_v, [v0, zeros_i])
    gy = plsc.load_gather(cont_v, [v0, ones_i])
    hx = plsc.load_gather(cont_v, [v1, zeros_i])
    hy = plsc.load_gather(cont_v, [v1, ones_i])
    ax = jnp.minimum(jnp.maximum(gx, 0.0), fmax)
    ay = jnp.minimum(jnp.maximum(gy, 0.0), fmax)
    bx = jnp.minimum(jnp.maximum(hx, 0.0), fmax)
    by = jnp.minimum(jnp.maximum(hy, 0.0), fmax)
    comp_v[pl.ds(0 * _E + e0, _LANES)] = ay
    comp_v[pl.ds(1 * _E + e0, _LANES)] = by
    comp_v[pl.ds(2 * _E + e0, _LANES)] = ax
    comp_v[pl.ds(3 * _E + e0, _LANES)] = bx - ax
    comp_v[pl.ds(4 * _E + e0, _LANES)] = 1.0 / (by - ay + 1e-9)
    ymin = jnp.minimum(ay, by)
    ymax = jnp.maximum(ay, by)
    ylo = jnp.maximum(_ceil_i32(ymin), ylo_half)
    yhi = jnp.minimum(_ceil_i32(ymax) - 1, ylo_half + _HHALF - 1)
    kf = lax.shift_right_arithmetic(ylo, 5)
    kl = lax.shift_right_arithmetic(yhi, 5)
    cnt = jnp.where(yhi >= ylo, jnp.maximum(kl - kf + 1, 0), 0)
    kfloc = kf - half * (_HHALF // 32)
    bnd_v[pl.ds(e0, _LANES)] = kfloc * _LANES + cnt
    return 0
  lax.fori_loop(0, _E // _LANES, pbody, 0)

  def zbody(r2, _):
    for dr in range(2):
      for cb in range(_W // _LANES):
        hist_v[r2 * 2 + dr, pl.ds(cb * _LANES, _LANES)] = zeros
    return 0
  lax.fori_loop(0, _HHALF // 2, zbody, 0)

  def ebody(e, _):
    es = jnp.full((_LANES,), e, jnp.int32)
    pk = jnp.max(plsc.load_gather(bnd_v, [es]))
    cnt = jnp.bitwise_and(pk, _LANES - 1)
    kf = lax.shift_right_arithmetic(pk, 4)

    @pl.when(cnt > 0)
    def _():
      ay = plsc.load_gather(comp_v, [es])
      by = plsc.load_gather(comp_v, [es + _E])
      ax = plsc.load_gather(comp_v, [es + 2 * _E])
      dx = plsc.load_gather(comp_v, [es + 3 * _E])
      inv = plsc.load_gather(comp_v, [es + 4 * _E])

      for i in range(4):
        live = i < cnt
        r0 = (kf + i) * 32 + lane
        for s in (0, _LANES):
          rowloc = r0 + s
          py = (rowloc + ylo_half).astype(jnp.float32)
          cond = (ay > py) != (by > py)
          xint = ax + ((py - ay) * inv) * dx
          c = _ceil_i32(xint)
          valid = cond & (c < _W) & live
          cc = jnp.minimum(jnp.maximum(c, 0), _W - 1)
          rsafe = jnp.bitwise_and(rowloc, _HHALF - 1)
          plsc.addupdate_scatter(hist_v, [rsafe, cc], ones, mask=valid)
    return 0
  lax.fori_loop(0, _E, ebody, 0)

  pltpu.sync_copy(hist_v, hist_hbm.at[p, pl.ds(half * _HHALF, _HHALF), :])


_sc_hist = functools.partial(
    pl.kernel,
    out_type=jax.ShapeDtypeStruct((_P, _H, _W), jnp.float32),
    mesh=plsc.VectorSubcoreMesh(core_axis_name="c", subcore_axis_name="s"),
    compiler_params=pltpu.CompilerParams(needs_layout_passes=False),
    scratch_types=[
        pltpu.VMEM((_E, 2), jnp.float32),
        pltpu.VMEM((5 * _E,), jnp.float32),
        pltpu.VMEM((_E + _LANES,), jnp.int32),
        pltpu.VMEM((_HHALF, _W), jnp.float32),
    ],
)(_sc_hist_body)


_YB = 128


def _tc_fuse_body(hist_ref, feat_ref, mask_ref, out_ref):
  hist = hist_ref[...]
  bi = lax.broadcasted_iota(jnp.int32, (_W, _W), 0)
  xi = lax.broadcasted_iota(jnp.int32, (_W, _W), 1)
  tri = (bi <= xi).astype(jnp.float32)
  cnt = jnp.dot(hist.reshape(_P * _YB, _W), tri,
                preferred_element_type=jnp.float32)
  par = cnt - 2.0 * jnp.floor(cnt * 0.5)
  mask = par.reshape(_P, _YB, _W)
  mask_ref[...] = mask
  mm = jnp.max(mask, axis=0)
  f = feat_ref[...]
  out_ref[...] = jnp.maximum(mm[None] * f + f, 0.0)


def kernel(contour, cnn_feature):
  bs, c_in, h, w = cnn_feature.shape
  hist = _sc_hist(contour)

  mask, fused = pl.pallas_call(
      _tc_fuse_body,
      grid=(_H // _YB,),
      in_specs=[
          pl.BlockSpec((_P, _YB, _W), lambda i: (0, i, 0)),
          pl.BlockSpec((c_in, _YB, _W), lambda i: (0, i, 0)),
      ],
      out_specs=[
          pl.BlockSpec((_P, _YB, _W), lambda i: (0, i, 0)),
          pl.BlockSpec((c_in, _YB, _W), lambda i: (0, i, 0)),
      ],
      out_shape=[
          jax.ShapeDtypeStruct((_P, _H, _W), jnp.float32),
          jax.ShapeDtypeStruct((c_in, _H, _W), jnp.float32),
      ],
  )(hist, cnn_feature[0])

  return mask[None], fused[None]


# speedup vs baseline: 1.0594x; 1.0594x over previous
import functools

import jax
import jax.numpy as jnp
from jax import lax
from jax.experimental import pallas as pl
from jax.experimental.pallas import tpu as pltpu
from jax.experimental.pallas import tpu_sc as plsc

_P = 16
_E = 128
_H = 256
_W = 256
_LANES = 16
_HHALF = _H // 2
_NCH = _HHALF // _LANES


def _ceil_i32(x):
  it = x.astype(jnp.int32)
  return it + (x > it.astype(jnp.float32)).astype(jnp.int32)


def _sc_hist_body(cont_hbm, hist_hbm, cont_v, comp_v, bnd_v, bnd2_v, hist_v):
  wid = lax.axis_index("s") * 2 + lax.axis_index("c")
  p = wid // 2
  half = wid % 2
  ylo_half = half * _HHALF

  pltpu.sync_copy(cont_hbm.at[p], cont_v)

  lane = lax.iota(jnp.int32, _LANES)
  ones = jnp.ones((_LANES,), jnp.float32)
  zeros = jnp.zeros((_LANES,), jnp.float32)
  zeros_i = jnp.zeros((_LANES,), jnp.int32)
  ones_i = jnp.ones((_LANES,), jnp.int32)
  fmax = float(_W - 1)

  bnd_v[pl.ds(_E, _LANES)] = zeros_i

  def pbody(j, _):
    e0 = pl.multiple_of(j * _LANES, _LANES)
    v0 = e0 + lane
    v1 = (v0 + 1) & (_E - 1)
    gx = plsc.load_gather(cont_v, [v0, zeros_i])
    gy = plsc.load_gather(cont_v, [v0, ones_i])
    hx = plsc.load_gather(cont_v, [v1, zeros_i])
    hy = plsc.load_gather(cont_v, [v1, ones_i])
    ax = jnp.minimum(jnp.maximum(gx, 0.0), fmax)
    ay = jnp.minimum(jnp.maximum(gy, 0.0), fmax)
    bx = jnp.minimum(jnp.maximum(hx, 0.0), fmax)
    by = jnp.minimum(jnp.maximum(hy, 0.0), fmax)
    comp_v[pl.ds(0 * _E + e0, _LANES)] = ay
    comp_v[pl.ds(1 * _E + e0, _LANES)] = by
    comp_v[pl.ds(2 * _E + e0, _LANES)] = ax
    comp_v[pl.ds(3 * _E + e0, _LANES)] = bx - ax
    comp_v[pl.ds(4 * _E + e0, _LANES)] = 1.0 / (by - ay + 1e-9)
    ymin = jnp.minimum(ay, by)
    ymax = jnp.maximum(ay, by)
    ylo = jnp.maximum(_ceil_i32(ymin), ylo_half)
    yhi = jnp.minimum(_ceil_i32(ymax) - 1, ylo_half + _HHALF - 1)
    kf = lax.shift_right_arithmetic(ylo, 5)
    kl = lax.shift_right_arithmetic(yhi, 5)
    cnt = jnp.where(yhi >= ylo, jnp.maximum(kl - kf + 1, 0), 0)
    kfloc = kf - half * (_HHALF // 32)
    bnd_v[pl.ds(e0, _LANES)] = kfloc * _LANES + cnt
    return 0
  lax.fori_loop(0, _E // _LANES, pbody, 0)

  def qbody(j, _):
    base = pl.multiple_of(j * _LANES, _LANES)
    idx0 = 2 * (base + lane)
    b0 = plsc.load_gather(bnd_v, [idx0])
    b1 = plsc.load_gather(bnd_v, [idx0 + 1])
    bnd2_v[pl.ds(base, _LANES)] = b0 | lax.shift_left(b1, 6)
    return 0
  lax.fori_loop(0, _E // (2 * _LANES), qbody, 0)

  def zbody(r2, _):
    for dr in range(2):
      for cb in range(_W // _LANES):
        hist_v[r2 * 2 + dr, pl.ds(cb * _LANES, _LANES)] = zeros
    return 0
  lax.fori_loop(0, _HHALF // 2, zbody, 0)

  def ebody(q, _):
    qs = jnp.full((_LANES,), q, jnp.int32)
    pk = jnp.max(plsc.load_gather(bnd2_v, [qs]))

    for sub in range(2):
      pke = pk if sub == 0 else lax.shift_right_logical(pk, 6)
      cnt = jnp.bitwise_and(pke, 15)
      kf = jnp.bitwise_and(lax.shift_right_logical(pke, 4), 3)
      es = qs * 2 + sub

      @pl.when(cnt > 0)
      def _():
        ay = plsc.load_gather(comp_v, [es])
        by = plsc.load_gather(comp_v, [es + _E])
        ax = plsc.load_gather(comp_v, [es + 2 * _E])
        dx = plsc.load_gather(comp_v, [es + 3 * _E])
        inv = plsc.load_gather(comp_v, [es + 4 * _E])

        def cbody(i, _):
          r0 = (kf + i) * 32 + lane
          for s in (0, _LANES):
            rowloc = r0 + s
            py = (rowloc + ylo_half).astype(jnp.float32)
            cond = (ay > py) != (by > py)
            xint = ax + ((py - ay) * inv) * dx
            c = _ceil_i32(xint)
            valid = cond & (c < _W)
            cc = jnp.minimum(jnp.maximum(c, 0), _W - 1)
            plsc.addupdate_scatter(hist_v, [rowloc, cc], ones, mask=valid)
          return 0
        lax.fori_loop(0, cnt, cbody, 0)
    return 0
  lax.fori_loop(0, _E // 2, ebody, 0)

  pltpu.sync_copy(hist_v, hist_hbm.at[p, pl.ds(half * _HHALF, _HHALF), :])


_sc_hist = functools.partial(
    pl.kernel,
    out_type=jax.ShapeDtypeStruct((_P, _H, _W), jnp.float32),
    mesh=plsc.VectorSubcoreMesh(core_axis_name="c", subcore_axis_name="s"),
    compiler_params=pltpu.CompilerParams(needs_layout_passes=False),
    scratch_types=[
        pltpu.VMEM((_E, 2), jnp.float32),
        pltpu.VMEM((5 * _E,), jnp.float32),
        pltpu.VMEM((_E + _LANES,), jnp.int32),
        pltpu.VMEM((_E // 2,), jnp.int32),
        pltpu.VMEM((_HHALF, _W), jnp.float32),
    ],
)(_sc_hist_body)


_YB = 128


def _tc_fuse_body(hist_ref, feat_ref, mask_ref, out_ref):
  hist = hist_ref[...]
  bi = lax.broadcasted_iota(jnp.int32, (_W, _W), 0)
  xi = lax.broadcasted_iota(jnp.int32, (_W, _W), 1)
  tri = (bi <= xi).astype(jnp.float32)
  cnt = jnp.dot(hist.reshape(_P * _YB, _W), tri,
                preferred_element_type=jnp.float32)
  par = cnt - 2.0 * jnp.floor(cnt * 0.5)
  mask = par.reshape(_P, _YB, _W)
  mask_ref[...] = mask
  mm = jnp.max(mask, axis=0)
  f = feat_ref[...]
  out_ref[...] = jnp.maximum(mm[None] * f + f, 0.0)


def kernel(contour, cnn_feature):
  bs, c_in, h, w = cnn_feature.shape
  hist = _sc_hist(contour)

  mask, fused = pl.pallas_call(
      _tc_fuse_body,
      grid=(_H // _YB,),
      in_specs=[
          pl.BlockSpec((_P, _YB, _W), lambda i: (0, i, 0)),
          pl.BlockSpec((c_in, _YB, _W), lambda i: (0, i, 0)),
      ],
      out_specs=[
          pl.BlockSpec((_P, _YB, _W), lambda i: (0, i, 0)),
          pl.BlockSpec((c_in, _YB, _W), lambda i: (0, i, 0)),
      ],
      out_shape=[
          jax.ShapeDtypeStruct((_P, _H, _W), jnp.float32),
          jax.ShapeDtypeStruct((c_in, _H, _W), jnp.float32),
      ],
  )(hist, cnn_feature[0])

  return mask[None], fused[None]


# speedup vs baseline: 1.0667x; 1.0070x over previous
import functools

import jax
import jax.numpy as jnp
from jax import lax
from jax.experimental import pallas as pl
from jax.experimental.pallas import tpu as pltpu
from jax.experimental.pallas import tpu_sc as plsc

_P = 16
_E = 128
_H = 256
_W = 256
_LANES = 16
_HHALF = _H // 2
_NCH = _HHALF // _LANES


def _ceil_i32(x):
  it = x.astype(jnp.int32)
  return it + (x > it.astype(jnp.float32)).astype(jnp.int32)


def _sc_hist_body(cont_hbm, hist_hbm, cont_v, comp_v, bnd_v, bnd2_v, hist_v):
  wid = lax.axis_index("s") * 2 + lax.axis_index("c")
  p = wid // 2
  half = wid % 2
  ylo_half = half * _HHALF

  pltpu.sync_copy(cont_hbm.at[p], cont_v)

  lane = lax.iota(jnp.int32, _LANES)
  ones = jnp.ones((_LANES,), jnp.float32)
  zeros = jnp.zeros((_LANES,), jnp.float32)
  zeros_i = jnp.zeros((_LANES,), jnp.int32)
  ones_i = jnp.ones((_LANES,), jnp.int32)
  fmax = float(_W - 1)

  bnd_v[pl.ds(_E, _LANES)] = zeros_i

  def pbody(j, _):
    e0 = pl.multiple_of(j * _LANES, _LANES)
    v0 = e0 + lane
    v1 = (v0 + 1) & (_E - 1)
    gx = plsc.load_gather(cont_v, [v0, zeros_i])
    gy = plsc.load_gather(cont_v, [v0, ones_i])
    hx = plsc.load_gather(cont_v, [v1, zeros_i])
    hy = plsc.load_gather(cont_v, [v1, ones_i])
    ax = jnp.minimum(jnp.maximum(gx, 0.0), fmax)
    ay = jnp.minimum(jnp.maximum(gy, 0.0), fmax)
    bx = jnp.minimum(jnp.maximum(hx, 0.0), fmax)
    by = jnp.minimum(jnp.maximum(hy, 0.0), fmax)
    comp_v[pl.ds(0 * _E + e0, _LANES)] = ay
    comp_v[pl.ds(1 * _E + e0, _LANES)] = by
    comp_v[pl.ds(2 * _E + e0, _LANES)] = ax
    comp_v[pl.ds(3 * _E + e0, _LANES)] = bx - ax
    comp_v[pl.ds(4 * _E + e0, _LANES)] = 1.0 / (by - ay + 1e-9)
    ymin = jnp.minimum(ay, by)
    ymax = jnp.maximum(ay, by)
    ylo = jnp.maximum(_ceil_i32(ymin), ylo_half)
    yhi = jnp.minimum(_ceil_i32(ymax) - 1, ylo_half + _HHALF - 1)
    kf = lax.shift_right_arithmetic(ylo, 5)
    kl = lax.shift_right_arithmetic(yhi, 5)
    cnt = jnp.where(yhi >= ylo, jnp.maximum(kl - kf + 1, 0), 0)
    kfloc = kf - half * (_HHALF // 32)
    bnd_v[pl.ds(e0, _LANES)] = kfloc * _LANES + cnt
    return 0
  lax.fori_loop(0, _E // _LANES, pbody, 0)

  def qbody(j, _):
    base = pl.multiple_of(j * _LANES, _LANES)
    idx0 = 4 * (base + lane)
    b0 = plsc.load_gather(bnd_v, [idx0])
    b1 = plsc.load_gather(bnd_v, [idx0 + 1])
    b2 = plsc.load_gather(bnd_v, [idx0 + 2])
    b3 = plsc.load_gather(bnd_v, [idx0 + 3])
    bnd2_v[pl.ds(base, _LANES)] = (
        b0 | lax.shift_left(b1, 6) | lax.shift_left(b2, 12)
        | lax.shift_left(b3, 18))
    return 0
  lax.fori_loop(0, _E // (4 * _LANES), qbody, 0)

  def zbody(r2, _):
    for dr in range(2):
      for cb in range(_W // _LANES):
        hist_v[r2 * 2 + dr, pl.ds(cb * _LANES, _LANES)] = zeros
    return 0
  lax.fori_loop(0, _HHALF // 2, zbody, 0)

  def ebody(q, _):
    qs = jnp.full((_LANES,), q, jnp.int32)
    pk = jnp.max(plsc.load_gather(bnd2_v, [qs]))

    for sub in range(4):
      pke = pk if sub == 0 else lax.shift_right_logical(pk, 6 * sub)
      cnt = jnp.bitwise_and(pke, 15)
      kf = jnp.bitwise_and(lax.shift_right_logical(pke, 4), 3)
      es = qs * 4 + sub

      @pl.when(cnt > 0)
      def _():
        ay = plsc.load_gather(comp_v, [es])
        by = plsc.load_gather(comp_v, [es + _E])
        ax = plsc.load_gather(comp_v, [es + 2 * _E])
        dx = plsc.load_gather(comp_v, [es + 3 * _E])
        inv = plsc.load_gather(comp_v, [es + 4 * _E])

        def cbody(i, _):
          r0 = (kf + i) * 32 + lane
          for s in (0, _LANES):
            rowloc = r0 + s
            py = (rowloc + ylo_half).astype(jnp.float32)
            cond = (ay > py) != (by > py)
            xint = ax + ((py - ay) * inv) * dx
            c = _ceil_i32(xint)
            valid = cond & (c < _W)
            cc = jnp.minimum(jnp.maximum(c, 0), _W - 1)
            plsc.addupdate_scatter(hist_v, [rowloc, cc], ones, mask=valid)
          return 0
        lax.fori_loop(0, cnt, cbody, 0)
    return 0
  lax.fori_loop(0, _E // 4, ebody, 0)

  pltpu.sync_copy(hist_v, hist_hbm.at[p, pl.ds(half * _HHALF, _HHALF), :])


_sc_hist = functools.partial(
    pl.kernel,
    out_type=jax.ShapeDtypeStruct((_P, _H, _W), jnp.float32),
    mesh=plsc.VectorSubcoreMesh(core_axis_name="c", subcore_axis_name="s"),
    compiler_params=pltpu.CompilerParams(needs_layout_passes=False),
    scratch_types=[
        pltpu.VMEM((_E, 2), jnp.float32),
        pltpu.VMEM((5 * _E,), jnp.float32),
        pltpu.VMEM((_E + _LANES,), jnp.int32),
        pltpu.VMEM((_E // 4,), jnp.int32),
        pltpu.VMEM((_HHALF, _W), jnp.float32),
    ],
)(_sc_hist_body)


_YB = 128


def _tc_fuse_body(hist_ref, feat_ref, mask_ref, out_ref):
  hist = hist_ref[...]
  bi = lax.broadcasted_iota(jnp.int32, (_W, _W), 0)
  xi = lax.broadcasted_iota(jnp.int32, (_W, _W), 1)
  tri = (bi <= xi).astype(jnp.float32)
  cnt = jnp.dot(hist.reshape(_P * _YB, _W), tri,
                preferred_element_type=jnp.float32)
  par = cnt - 2.0 * jnp.floor(cnt * 0.5)
  mask = par.reshape(_P, _YB, _W)
  mask_ref[...] = mask
  mm = jnp.max(mask, axis=0)
  f = feat_ref[...]
  out_ref[...] = jnp.maximum(mm[None] * f + f, 0.0)


def kernel(contour, cnn_feature):
  bs, c_in, h, w = cnn_feature.shape
  hist = _sc_hist(contour)

  mask, fused = pl.pallas_call(
      _tc_fuse_body,
      grid=(_H // _YB,),
      in_specs=[
          pl.BlockSpec((_P, _YB, _W), lambda i: (0, i, 0)),
          pl.BlockSpec((c_in, _YB, _W), lambda i: (0, i, 0)),
      ],
      out_specs=[
          pl.BlockSpec((_P, _YB, _W), lambda i: (0, i, 0)),
          pl.BlockSpec((c_in, _YB, _W), lambda i: (0, i, 0)),
      ],
      out_shape=[
          jax.ShapeDtypeStruct((_P, _H, _W), jnp.float32),
          jax.ShapeDtypeStruct((c_in, _H, _W), jnp.float32),
      ],
  )(hist, cnn_feature[0])

  return mask[None], fused[None]


# speedup vs baseline: 1.0726x; 1.0055x over previous
import functools

import jax
import jax.numpy as jnp
from jax import lax
from jax.experimental import pallas as pl
from jax.experimental.pallas import tpu as pltpu
from jax.experimental.pallas import tpu_sc as plsc

_P = 16
_E = 128
_H = 256
_W = 256
_LANES = 16
_HHALF = _H // 2


def _ceil_i32(x):
  it = x.astype(jnp.int32)
  return it + (x > it.astype(jnp.float32)).astype(jnp.int32)


def _sc_hist_body(cont_hbm, hist_hbm, cont_v, comp_v, bnd_v, bnd2_v, hist_v):
  wid = lax.axis_index("s") * 2 + lax.axis_index("c")
  p = wid // 2
  half = wid % 2
  ylo_half = half * _HHALF

  pltpu.sync_copy(cont_hbm.at[p], cont_v)

  lane = lax.iota(jnp.int32, _LANES)
  ones = jnp.ones((_LANES,), jnp.float32)
  zeros = jnp.zeros((_LANES,), jnp.float32)
  zeros_i = jnp.zeros((_LANES,), jnp.int32)
  ones_i = jnp.ones((_LANES,), jnp.int32)
  fmax = float(_W - 1)

  def pbody(j, _):
    e0 = pl.multiple_of(j * _LANES, _LANES)
    v0 = e0 + lane
    v1 = (v0 + 1) & (_E - 1)
    gx = plsc.load_gather(cont_v, [v0, zeros_i])
    gy = plsc.load_gather(cont_v, [v0, ones_i])
    hx = plsc.load_gather(cont_v, [v1, zeros_i])
    hy = plsc.load_gather(cont_v, [v1, ones_i])
    ax = jnp.minimum(jnp.maximum(gx, 0.0), fmax)
    ay = jnp.minimum(jnp.maximum(gy, 0.0), fmax)
    bx = jnp.minimum(jnp.maximum(hx, 0.0), fmax)
    by = jnp.minimum(jnp.maximum(hy, 0.0), fmax)
    comp_v[pl.ds(0 * _E + e0, _LANES)] = ay
    comp_v[pl.ds(1 * _E + e0, _LANES)] = by
    comp_v[pl.ds(2 * _E + e0, _LANES)] = ax
    comp_v[pl.ds(3 * _E + e0, _LANES)] = bx - ax
    comp_v[pl.ds(4 * _E + e0, _LANES)] = 1.0 / (by - ay + 1e-9)
    ymin = jnp.minimum(ay, by)
    ymax = jnp.maximum(ay, by)
    ylo = jnp.maximum(_ceil_i32(ymin), ylo_half)
    yhi = jnp.minimum(_ceil_i32(ymax) - 1, ylo_half + _HHALF - 1)
    kf = lax.shift_right_arithmetic(ylo, 5)
    kl = lax.shift_right_arithmetic(yhi, 5)
    cnt = jnp.where(yhi >= ylo, jnp.maximum(kl - kf + 1, 0), 0)
    kfloc = jnp.bitwise_and(kf - half * (_HHALF // 32), 3)
    bnd_v[pl.ds(e0, _LANES)] = kfloc * _LANES + cnt
    return 0
  lax.fori_loop(0, _E // _LANES, pbody, 0)

  def qbody(j, _):
    base = pl.multiple_of(j * _LANES, _LANES)
    idx0 = 4 * (base + lane)
    b0 = plsc.load_gather(bnd_v, [idx0])
    b1 = plsc.load_gather(bnd_v, [idx0 + 1])
    b2 = plsc.load_gather(bnd_v, [idx0 + 2])
    b3 = plsc.load_gather(bnd_v, [idx0 + 3])
    bnd2_v[pl.ds(base, _LANES)] = (
        b0 | lax.shift_left(b1, 6) | lax.shift_left(b2, 12)
        | lax.shift_left(b3, 18))
    return 0
  lax.fori_loop(0, _E // (4 * _LANES), qbody, 0)

  def zbody(r2, _):
    for dr in range(2):
      for cb in range(_W // _LANES):
        hist_v[r2 * 2 + dr, pl.ds(cb * _LANES, _LANES)] = zeros
    return 0
  lax.fori_loop(0, _HHALF // 2, zbody, 0)

  def ebody(q, _):
    qs = jnp.full((_LANES,), q, jnp.int32)
    pk = jnp.max(plsc.load_gather(bnd2_v, [qs]))

    for sub in range(4):
      pke = pk if sub == 0 else lax.shift_right_logical(pk, 6 * sub)
      cnt = jnp.bitwise_and(pke, 15)
      kf = jnp.bitwise_and(lax.shift_right_logical(pke, 4), 3)
      es = qs * 4 + sub

      @pl.when(cnt > 0)
      def _():
        ay = plsc.load_gather(comp_v, [es])
        by = plsc.load_gather(comp_v, [es + _E])
        ax = plsc.load_gather(comp_v, [es + 2 * _E])
        dx = plsc.load_gather(comp_v, [es + 3 * _E])
        inv = plsc.load_gather(comp_v, [es + 4 * _E])

        def cbody(i, _):
          r0 = (kf + i) * 32 + lane
          for s in (0, _LANES):
            rowloc = r0 + s
            py = (rowloc + ylo_half).astype(jnp.float32)
            cond = (ay > py) != (by > py)
            xint = ax + ((py - ay) * inv) * dx
            c = _ceil_i32(xint)
            valid = cond & (c < _W)
            cc = jnp.minimum(jnp.maximum(c, 0), _W - 1)
            plsc.addupdate_scatter(hist_v, [rowloc, cc], ones, mask=valid)
          return 0
        lax.fori_loop(0, cnt, cbody, 0)
    return 0
  lax.fori_loop(0, _E // 4, ebody, 0)

  pltpu.sync_copy(hist_v, hist_hbm.at[p, pl.ds(half * _HHALF, _HHALF), :])


_sc_hist = functools.partial(
    pl.kernel,
    out_type=jax.ShapeDtypeStruct((_P, _H, _W), jnp.float32),
    mesh=plsc.VectorSubcoreMesh(core_axis_name="c", subcore_axis_name="s"),
    compiler_params=pltpu.CompilerParams(needs_layout_passes=False),
    scratch_types=[
        pltpu.VMEM((_E, 2), jnp.float32),
        pltpu.VMEM((5 * _E,), jnp.float32),
        pltpu.VMEM((_E,), jnp.int32),
        pltpu.VMEM((_E // 4,), jnp.int32),
        pltpu.VMEM((_HHALF, _W), jnp.float32),
    ],
)(_sc_hist_body)


_YB = 128


def _tc_fuse_body(hist_ref, feat_ref, mask_ref, out_ref):
  hist = hist_ref[...]
  bi = lax.broadcasted_iota(jnp.int32, (_W, _W), 0)
  xi = lax.broadcasted_iota(jnp.int32, (_W, _W), 1)
  tri = (bi <= xi).astype(jnp.float32)
  cnt = jnp.dot(hist.reshape(_P * _YB, _W), tri,
                preferred_element_type=jnp.float32)
  par = cnt - 2.0 * jnp.floor(cnt * 0.5)
  mask = par.reshape(_P, _YB, _W)
  mask_ref[...] = mask
  mm = jnp.max(mask, axis=0)
  f = feat_ref[...]
  out_ref[...] = jnp.maximum(mm[None] * f + f, 0.0)


def kernel(contour, cnn_feature):
  bs, c_in, h, w = cnn_feature.shape
  hist = _sc_hist(contour)

  mask, fused = pl.pallas_call(
      _tc_fuse_body,
      grid=(_H // _YB,),
      in_specs=[
          pl.BlockSpec((_P, _YB, _W), lambda i: (0, i, 0)),
          pl.BlockSpec((c_in, _YB, _W), lambda i: (0, i, 0)),
      ],
      out_specs=[
          pl.BlockSpec((_P, _YB, _W), lambda i: (0, i, 0)),
          pl.BlockSpec((c_in, _YB, _W), lambda i: (0, i, 0)),
      ],
      out_shape=[
          jax.ShapeDtypeStruct((_P, _H, _W), jnp.float32),
          jax.ShapeDtypeStruct((c_in, _H, _W), jnp.float32),
      ],
  )(hist, cnn_feature[0])

  return mask[None], fused[None]
